# Initial kernel scaffold; baseline (speedup 1.0000x reference)
import jax
import jax.numpy as jnp
from jax.experimental import pallas as pl


def kernel(x, edge_index, W, b):
    raise NotImplementedError("write your pallas kernel here")


# speedup vs baseline: 27.4838x; 27.4838x over previous
import dataclasses
import functools

import jax
import jax.numpy as jnp
from jax import lax
from jax.experimental import pallas as pl
from jax.experimental.pallas import tpu as pltpu
from jax.experimental.pallas import tpu_sc as plsc

N = 10000
E = 320000
D = 128

NC = 2
NS = 16
NW = NC * NS
CH = 80
EPT = E // NW
NCHUNK = EPT // CH
ROWS_PT = N // NS
BLK = 1000

_mesh = plsc.VectorSubcoreMesh(
    core_axis_name="c", subcore_axis_name="s", num_cores=NC, num_subcores=NS
)

_sc_params = pltpu.CompilerParams()
if "needs_layout_passes" in pltpu.CompilerParams.__dataclass_fields__:
    _sc_params = dataclasses.replace(_sc_params, needs_layout_passes=False)


@functools.partial(
    pl.kernel,
    out_type=jax.ShapeDtypeStruct((NW, N), jnp.float32),
    mesh=_mesh,
    scratch_types=[
        pltpu.VMEM((N,), jnp.float32),
        pltpu.VMEM((NCHUNK, CH), jnp.int32),
    ],
    compiler_params=_sc_params,
)
def _deg_kernel(dst_hbm, degp_hbm, hist, idx2d):
    wid = lax.axis_index("s") * NC + lax.axis_index("c")
    zeros16 = jnp.zeros((16,), jnp.float32)

    @pl.loop(0, N // 16)
    def _(j):
        hist[pl.ds(j * 16, 16)] = zeros16

    pltpu.sync_copy(dst_hbm.at[wid], idx2d)
    ones16 = jnp.ones((16,), jnp.float32)

    @pl.loop(0, NCHUNK)
    def _(j):
        @pl.loop(0, CH // 16)
        def _(c):
            v = idx2d.at[j][pl.ds(c * 16, 16)]
            plsc.addupdate_scatter(hist, [v], ones16)

    pltpu.sync_copy(hist, degp_hbm.at[wid])


@functools.partial(
    pl.kernel,
    out_type=jax.ShapeDtypeStruct((NC, NS, ROWS_PT, D), jnp.float32),
    mesh=_mesh,
    scratch_types=[
        pltpu.VMEM_SHARED((N, D), jnp.float32),
        pltpu.VMEM((NCHUNK, CH), jnp.int32),
        pltpu.VMEM((NCHUNK, CH), jnp.int32),
        pltpu.VMEM((CH, D), jnp.float32),
    ],
    compiler_params=_sc_params,
)
def _edge_kernel(src_hbm, dst_hbm, g_hbm, parts_hbm, acc, sidx, didx, rows):
    cid = lax.axis_index("c")
    sid = lax.axis_index("s")
    zeros16 = jnp.zeros((16,), jnp.float32)

    @pl.loop(0, CH)
    def _(r):
        @pl.loop(0, D // 16)
        def _(c):
            rows.at[r][pl.ds(c * 16, 16)] = zeros16

    @pl.loop(0, ROWS_PT // CH)
    def _(i):
        pltpu.sync_copy(rows, acc.at[pl.ds(sid * ROWS_PT + i * CH, CH)])

    _rem = ROWS_PT - (ROWS_PT // CH) * CH
    pltpu.sync_copy(
        rows.at[pl.ds(0, _rem)],
        acc.at[pl.ds(sid * ROWS_PT + (ROWS_PT // CH) * CH, _rem)],
    )

    plsc.subcore_barrier()

    widx = cid * NS + sid
    pltpu.sync_copy(src_hbm.at[widx], sidx)
    pltpu.sync_copy(dst_hbm.at[widx], didx)

    @pl.loop(0, NCHUNK)
    def _(j):
        pltpu.sync_copy(g_hbm.at[sidx.at[j]], rows)
        pltpu.sync_copy(rows, acc.at[didx.at[j]], add=True)

    plsc.subcore_barrier()
    pltpu.sync_copy(
        acc.at[pl.ds(sid * ROWS_PT, ROWS_PT)],
        parts_hbm.at[cid].at[sid],
    )


def _gw_body(x_ref, w_ref, degp_ref, g_ref):
    deg = jnp.sum(degp_ref[...], axis=1) + 1.0
    dinv = lax.rsqrt(deg)
    h = jnp.dot(x_ref[...], w_ref[...], preferred_element_type=jnp.float32)
    g_ref[...] = h * dinv[:, None]


def _final_body(p_ref, g_ref, degp_ref, b_ref, o_ref):
    deg = jnp.sum(degp_ref[...], axis=1) + 1.0
    dinv = lax.rsqrt(deg)
    p = p_ref[...]
    s = p[0] + p[1] + g_ref[...]
    o_ref[...] = jnp.maximum(s * dinv[:, None] + b_ref[...], 0.0)


def kernel(x, edge_index, W, b):
    src = edge_index[0].reshape(NW, NCHUNK, CH)
    dst = edge_index[1].reshape(NW, NCHUNK, CH)

    degp = _deg_kernel(dst).T

    g = pl.pallas_call(
        _gw_body,
        grid=(N // BLK,),
        in_specs=[
            pl.BlockSpec((BLK, D), lambda i: (i, 0)),
            pl.BlockSpec((D, D), lambda i: (0, 0)),
            pl.BlockSpec((BLK, NW), lambda i: (i, 0)),
        ],
        out_specs=pl.BlockSpec((BLK, D), lambda i: (i, 0)),
        out_shape=jax.ShapeDtypeStruct((N, D), jnp.float32),
    )(x, W, degp)

    parts = _edge_kernel(src, dst, g).reshape(NC, N, D)

    out = pl.pallas_call(
        _final_body,
        grid=(N // BLK,),
        in_specs=[
            pl.BlockSpec((NC, BLK, D), lambda i: (0, i, 0)),
            pl.BlockSpec((BLK, D), lambda i: (i, 0)),
            pl.BlockSpec((BLK, NW), lambda i: (i, 0)),
            pl.BlockSpec((1, D), lambda i: (0, 0)),
        ],
        out_specs=pl.BlockSpec((BLK, D), lambda i: (i, 0)),
        out_shape=jax.ShapeDtypeStruct((N, D), jnp.float32),
    )(parts, g, degp, b.reshape(1, D))

    return out


# speedup vs baseline: 36.6631x; 1.3340x over previous
import dataclasses
import functools

import jax
import jax.numpy as jnp
from jax import lax
from jax.experimental import pallas as pl
from jax.experimental.pallas import tpu as pltpu
from jax.experimental.pallas import tpu_sc as plsc

N = 10000
E = 320000
D = 128

NC = 2
NS = 16
NW = NC * NS
CH = 128
NCH_TOT = E // CH
CPT = 80
PAD_ROWS = NW * CPT
HW = CPT // 2
NB = 2
ROWS_PT = N // NS
BLK = 1000

_mesh = plsc.VectorSubcoreMesh(
    core_axis_name="c", subcore_axis_name="s", num_cores=NC, num_subcores=NS
)

_sc_params = pltpu.CompilerParams()
if "needs_layout_passes" in pltpu.CompilerParams.__dataclass_fields__:
    _sc_params = dataclasses.replace(_sc_params, needs_layout_passes=False)


@functools.partial(
    pl.kernel,
    out_type=jax.ShapeDtypeStruct((NW, N), jnp.float32),
    mesh=_mesh,
    scratch_types=[
        pltpu.VMEM((N,), jnp.float32),
        pltpu.VMEM((CPT, CH), jnp.int32),
    ],
    compiler_params=_sc_params,
)
def _deg_kernel(dst_hbm, degp_hbm, hist, idx2d):
    wid = lax.axis_index("s") * NC + lax.axis_index("c")
    zeros16 = jnp.zeros((16,), jnp.float32)

    @pl.loop(0, N // 16)
    def _(j):
        hist[pl.ds(j * 16, 16)] = zeros16

    base = wid * CPT
    nch = jnp.minimum(CPT, NCH_TOT - base)
    pltpu.sync_copy(dst_hbm.at[pl.ds(base, CPT)], idx2d)
    ones16 = jnp.ones((16,), jnp.float32)

    @pl.loop(0, nch)
    def _(j):
        @pl.loop(0, CH // 16)
        def _(c):
            v = idx2d.at[j][pl.ds(c * 16, 16)]
            plsc.addupdate_scatter(hist, [v], ones16)

    pltpu.sync_copy(hist, degp_hbm.at[wid])


@functools.partial(
    pl.kernel,
    out_type=jax.ShapeDtypeStruct((NC, NS, ROWS_PT, D), jnp.float32),
    mesh=_mesh,
    scratch_types=[
        pltpu.VMEM_SHARED((N, D), jnp.float32),
        pltpu.VMEM((CPT, CH), jnp.int32),
        pltpu.VMEM((HW, CH), jnp.int32),
        pltpu.VMEM((NB, CH, D), jnp.float32),
        pltpu.SemaphoreType.DMA((NB,)),
        pltpu.SemaphoreType.DMA((NB,)),
    ],
    compiler_params=_sc_params,
)
def _edge_kernel(src_hbm, dst_hbm, g_hbm, parts_hbm, acc, sidx, didx, rows, gsem, ssem):
    cid = lax.axis_index("c")
    sid = lax.axis_index("s")
    widx = cid * NS + sid
    zeros16 = jnp.zeros((16,), jnp.float32)
    r0 = rows.at[0]

    @pl.loop(0, CH)
    def _(r):
        @pl.loop(0, D // 16)
        def _(c):
            r0.at[r][pl.ds(c * 16, 16)] = zeros16

    @pl.loop(0, ROWS_PT // CH)
    def _(i):
        pltpu.sync_copy(r0, acc.at[pl.ds(sid * ROWS_PT + i * CH, CH)])

    pltpu.sync_copy(
        r0.at[pl.ds(0, ROWS_PT % CH)],
        acc.at[pl.ds(sid * ROWS_PT + (ROWS_PT // CH) * CH, ROWS_PT % CH)],
    )

    plsc.subcore_barrier()

    base = widx * CPT
    nch = jnp.minimum(CPT, NCH_TOT - base)
    pltpu.sync_copy(src_hbm.at[pl.ds(base, CPT)], sidx)

    for h in range(2):
        nh = jnp.clip(nch - h * HW, 0, HW)

        @pl.when(nh > 0)
        def _():
            pltpu.sync_copy(dst_hbm.at[pl.ds(base + h * HW, HW)], didx)
            for b in range(NB):
                @pl.when(b < nh)
                def _():
                    pltpu.async_copy(
                        g_hbm.at[sidx.at[h * HW + b]], rows.at[b], gsem.at[b]
                    )

            @pl.loop(0, nh // NB)
            def _(p):
                for b in range(NB):
                    j = p * NB + b
                    pltpu.make_async_copy(
                        g_hbm.at[sidx.at[h * HW + j]], rows.at[b], gsem.at[b]
                    ).wait()
                    pltpu.async_copy(
                        rows.at[b], acc.at[didx.at[j]], ssem.at[b], add=True
                    )
                for b in range(NB):
                    j = p * NB + b
                    pltpu.make_async_copy(
                        rows.at[b], acc.at[didx.at[j]], ssem.at[b]
                    ).wait()

                    @pl.when(j + NB < nh)
                    def _():
                        pltpu.async_copy(
                            g_hbm.at[sidx.at[h * HW + j + NB]], rows.at[b], gsem.at[b]
                        )

    plsc.subcore_barrier()
    pltpu.sync_copy(
        acc.at[pl.ds(sid * ROWS_PT, ROWS_PT)],
        parts_hbm.at[cid].at[sid],
    )


def _gw_body(x_ref, w_ref, degp_ref, g_ref):
    deg = jnp.sum(degp_ref[...], axis=1) + 1.0
    dinv = lax.rsqrt(deg)
    h = jnp.dot(x_ref[...], w_ref[...], preferred_element_type=jnp.float32)
    g_ref[...] = h * dinv[:, None]


def _final_body(p_ref, g_ref, degp_ref, b_ref, o_ref):
    deg = jnp.sum(degp_ref[...], axis=1) + 1.0
    dinv = lax.rsqrt(deg)
    p = p_ref[...]
    s = p[0] + p[1] + g_ref[...]
    o_ref[...] = jnp.maximum(s * dinv[:, None] + b_ref[...], 0.0)


def kernel(x, edge_index, W, b):
    pad = jnp.zeros((2, PAD_ROWS * CH - E), jnp.int32)
    ei = jnp.concatenate([edge_index, pad], axis=1)
    src = ei[0].reshape(PAD_ROWS, CH)
    dst = ei[1].reshape(PAD_ROWS, CH)

    degp = _deg_kernel(dst).T

    g = pl.pallas_call(
        _gw_body,
        grid=(N // BLK,),
        in_specs=[
            pl.BlockSpec((BLK, D), lambda i: (i, 0)),
            pl.BlockSpec((D, D), lambda i: (0, 0)),
            pl.BlockSpec((BLK, NW), lambda i: (i, 0)),
        ],
        out_specs=pl.BlockSpec((BLK, D), lambda i: (i, 0)),
        out_shape=jax.ShapeDtypeStruct((N, D), jnp.float32),
    )(x, W, degp)

    parts = _edge_kernel(src, dst, g).reshape(NC, N, D)

    out = pl.pallas_call(
        _final_body,
        grid=(N // BLK,),
        in_specs=[
            pl.BlockSpec((NC, BLK, D), lambda i: (0, i, 0)),
            pl.BlockSpec((BLK, D), lambda i: (i, 0)),
            pl.BlockSpec((BLK, NW), lambda i: (i, 0)),
            pl.BlockSpec((1, D), lambda i: (0, 0)),
        ],
        out_specs=pl.BlockSpec((BLK, D), lambda i: (i, 0)),
        out_shape=jax.ShapeDtypeStruct((N, D), jnp.float32),
    )(parts, g, degp, b.reshape(1, D))

    return out


# speedup vs baseline: 41.1136x; 1.1214x over previous
import dataclasses
import functools

import jax
import jax.numpy as jnp
from jax import lax
from jax.experimental import pallas as pl
from jax.experimental.pallas import tpu as pltpu
from jax.experimental.pallas import tpu_sc as plsc

N = 10000
E = 320000
D = 128

NC = 2
NS = 16
NW = NC * NS
JUNK = 48
N2 = N + JUNK
CH = 64
CPT = 160
EPT = CPT * CH
PAD_E = NW * EPT
GRP = 8
NGRP = CPT // GRP
NB = 4
ROWS_PT = N // NS
ROWS_Z = N2 // NS
K1CPT = PAD_E // NW // 128
BLK = 1000

_mesh = plsc.VectorSubcoreMesh(
    core_axis_name="c", subcore_axis_name="s", num_cores=NC, num_subcores=NS
)

_sc_params = pltpu.CompilerParams()
if "needs_layout_passes" in pltpu.CompilerParams.__dataclass_fields__:
    _sc_params = dataclasses.replace(_sc_params, needs_layout_passes=False)


@functools.partial(
    pl.kernel,
    out_type=jax.ShapeDtypeStruct((NW, N2), jnp.float32),
    mesh=_mesh,
    scratch_types=[
        pltpu.VMEM((N2,), jnp.float32),
        pltpu.VMEM((K1CPT, 128), jnp.int32),
    ],
    compiler_params=_sc_params,
)
def _deg_kernel(dst_hbm, degp_hbm, hist, idx2d):
    wid = lax.axis_index("s") * NC + lax.axis_index("c")
    zeros16 = jnp.zeros((16,), jnp.float32)

    @pl.loop(0, N2 // 16)
    def _(j):
        hist[pl.ds(j * 16, 16)] = zeros16

    pltpu.sync_copy(dst_hbm.at[pl.ds(wid * K1CPT, K1CPT)], idx2d)
    ones16 = jnp.ones((16,), jnp.float32)

    @pl.loop(0, K1CPT)
    def _(j):
        @pl.loop(0, 128 // 16)
        def _(c):
            v = idx2d.at[j][pl.ds(c * 16, 16)]
            plsc.addupdate_scatter(hist, [v], ones16)

    pltpu.sync_copy(hist, degp_hbm.at[wid])


@functools.partial(
    pl.kernel,
    out_type=jax.ShapeDtypeStruct((NC, NS, ROWS_PT, D), jnp.float32),
    mesh=_mesh,
    scratch_types=[
        pltpu.VMEM_SHARED((N2, D), jnp.float32),
        pltpu.VMEM((EPT,), jnp.int32),
        pltpu.VMEM((2, GRP, CH), jnp.int32),
        pltpu.VMEM((NB, CH, D), jnp.float32),
        pltpu.SemaphoreType.DMA((NB,)),
        pltpu.SemaphoreType.DMA((NB,)),
        pltpu.SemaphoreType.DMA((2,)),
    ],
    compiler_params=_sc_params,
)
def _edge_kernel(src_hbm, dst_hbm, g_hbm, parts_hbm, acc, sidx, didx, rows,
                 gsem, ssem, isem):
    cid = lax.axis_index("c")
    sid = lax.axis_index("s")
    widx = cid * NS + sid
    zeros16 = jnp.zeros((16,), jnp.float32)
    r0 = rows.at[0]

    @pl.loop(0, CH)
    def _(r):
        @pl.loop(0, D // 16)
        def _(c):
            r0.at[r][pl.ds(c * 16, 16)] = zeros16

    @pl.loop(0, ROWS_Z // CH)
    def _(i):
        pltpu.sync_copy(r0, acc.at[pl.ds(sid * ROWS_Z + i * CH, CH)])

    pltpu.sync_copy(
        r0.at[pl.ds(0, ROWS_Z % CH)],
        acc.at[pl.ds(sid * ROWS_Z + (ROWS_Z // CH) * CH, ROWS_Z % CH)],
    )

    plsc.subcore_barrier()

    tb = widx * CPT
    pltpu.sync_copy(src_hbm.at[pl.ds(widx * EPT, EPT)], sidx)

    for q in range(2):
        pltpu.async_copy(dst_hbm.at[pl.ds(tb + q * GRP, GRP)], didx.at[q],
                         isem.at[q])
    for c in range(NB):
        pltpu.async_copy(g_hbm.at[sidx.at[pl.ds(c * CH, CH)]], rows.at[c],
                         gsem.at[c])

    @pl.loop(0, NGRP // 2)
    def _(s):
        for q in range(2):
            g = s * 2 + q
            grow = tb + g * GRP
            pltpu.make_async_copy(
                dst_hbm.at[pl.ds(grow, GRP)], didx.at[q], isem.at[q]
            ).wait()
            for r in range(GRP // NB):
                for c in range(NB):
                    k = g * GRP + r * NB + c
                    pltpu.make_async_copy(
                        g_hbm.at[sidx.at[pl.ds(k * CH, CH)]], rows.at[c],
                        gsem.at[c],
                    ).wait()
                    pltpu.async_copy(
                        rows.at[c], acc.at[didx.at[q].at[r * NB + c]],
                        ssem.at[c], add=True,
                    )
                for c in range(NB):
                    k = g * GRP + r * NB + c
                    pltpu.make_async_copy(
                        rows.at[c], acc.at[didx.at[q].at[r * NB + c]],
                        ssem.at[c],
                    ).wait()

                    @pl.when(k + NB < CPT)
                    def _():
                        pltpu.async_copy(
                            g_hbm.at[sidx.at[pl.ds((k + NB) * CH, CH)]],
                            rows.at[c], gsem.at[c],
                        )

            @pl.when(g + 2 < NGRP)
            def _():
                pltpu.async_copy(
                    dst_hbm.at[pl.ds(grow + 2 * GRP, GRP)], didx.at[q],
                    isem.at[q],
                )

    plsc.subcore_barrier()
    pltpu.sync_copy(
        acc.at[pl.ds(sid * ROWS_PT, ROWS_PT)],
        parts_hbm.at[cid].at[sid],
    )


def _gw_body(x_ref, w_ref, degp_ref, g_ref):
    deg = jnp.sum(degp_ref[...], axis=1) + 1.0
    dinv = lax.rsqrt(deg)
    h = jnp.dot(x_ref[...], w_ref[...], preferred_element_type=jnp.float32)
    g_ref[...] = h * dinv[:, None]


def _final_body(p_ref, g_ref, degp_ref, b_ref, o_ref):
    deg = jnp.sum(degp_ref[...], axis=1) + 1.0
    dinv = lax.rsqrt(deg)
    p = p_ref[...]
    s = p[0] + p[1] + g_ref[...]
    o_ref[...] = jnp.maximum(s * dinv[:, None] + b_ref[...], 0.0)


def kernel(x, edge_index, W, b):
    npad = PAD_E - E
    ar = jnp.arange(npad, dtype=jnp.int32)
    src_flat = jnp.concatenate([edge_index[0], ar % N])
    dst_flat = jnp.concatenate([edge_index[1], ar % JUNK + N])
    dst64 = dst_flat.reshape(PAD_E // CH, CH)
    dst128 = dst_flat.reshape(PAD_E // 128, 128)

    degp = _deg_kernel(dst128)[:, :N].T

    g = pl.pallas_call(
        _gw_body,
        grid=(N // BLK,),
        in_specs=[
            pl.BlockSpec((BLK, D), lambda i: (i, 0)),
            pl.BlockSpec((D, D), lambda i: (0, 0)),
            pl.BlockSpec((BLK, NW), lambda i: (i, 0)),
        ],
        out_specs=pl.BlockSpec((BLK, D), lambda i: (i, 0)),
        out_shape=jax.ShapeDtypeStruct((N, D), jnp.float32),
    )(x, W, degp)

    parts = _edge_kernel(src_flat, dst64, g).reshape(NC, N, D)

    out = pl.pallas_call(
        _final_body,
        grid=(N // BLK,),
        in_specs=[
            pl.BlockSpec((NC, BLK, D), lambda i: (0, i, 0)),
            pl.BlockSpec((BLK, D), lambda i: (i, 0)),
            pl.BlockSpec((BLK, NW), lambda i: (i, 0)),
            pl.BlockSpec((1, D), lambda i: (0, 0)),
        ],
        out_specs=pl.BlockSpec((BLK, D), lambda i: (i, 0)),
        out_shape=jax.ShapeDtypeStruct((N, D), jnp.float32),
    )(parts, g, degp, b.reshape(1, D))

    return out


# speedup vs baseline: 43.9772x; 1.0697x over previous
import dataclasses
import functools

import jax
import jax.numpy as jnp
from jax import lax
from jax.experimental import pallas as pl
from jax.experimental.pallas import tpu as pltpu
from jax.experimental.pallas import tpu_sc as plsc

N = 10000
E = 320000
D = 128

NC = 2
NS = 16
NW = NC * NS
CH = 64
NCH = E // CH
CPT = 160
EPT = CPT * CH
GRP = 8
NGRP = CPT // GRP
NB = 4
ROWS_PT = N // NS
CO = 624
K1CH = E // 128
K1CPT = 80
BLK = 2000

_mesh = plsc.VectorSubcoreMesh(
    core_axis_name="c", subcore_axis_name="s", num_cores=NC, num_subcores=NS
)

_sc_params = pltpu.CompilerParams()
if "needs_layout_passes" in pltpu.CompilerParams.__dataclass_fields__:
    _sc_params = dataclasses.replace(_sc_params, needs_layout_passes=False)


@functools.partial(
    pl.kernel,
    out_type=jax.ShapeDtypeStruct((NW, N), jnp.float32),
    mesh=_mesh,
    scratch_types=[
        pltpu.VMEM((N,), jnp.float32),
        pltpu.VMEM((K1CPT, 128), jnp.int32),
    ],
    compiler_params=_sc_params,
)
def _deg_kernel(dst_hbm, degp_hbm, hist, idx2d):
    wid = lax.axis_index("s") * NC + lax.axis_index("c")
    zeros16 = jnp.zeros((16,), jnp.float32)

    @pl.loop(0, N // 16)
    def _(j):
        hist[pl.ds(j * 16, 16)] = zeros16

    nch = jnp.minimum(K1CPT, K1CH - wid * K1CPT)

    @pl.when(wid < NW - 1)
    def _():
        pltpu.sync_copy(dst_hbm.at[pl.ds(wid * K1CPT, K1CPT)], idx2d)

    @pl.when(wid == NW - 1)
    def _():
        pltpu.sync_copy(
            dst_hbm.at[pl.ds((NW - 1) * K1CPT, K1CH - (NW - 1) * K1CPT)],
            idx2d.at[pl.ds(0, K1CH - (NW - 1) * K1CPT)],
        )

    ones16 = jnp.ones((16,), jnp.float32)

    @pl.loop(0, nch)
    def _(j):
        @pl.loop(0, 128 // 16)
        def _(c):
            v = idx2d.at[j][pl.ds(c * 16, 16)]
            plsc.addupdate_scatter(hist, [v], ones16)

    pltpu.sync_copy(hist, degp_hbm.at[wid])


@functools.partial(
    pl.kernel,
    out_type=jax.ShapeDtypeStruct((NC, N, D), jnp.float32),
    mesh=_mesh,
    scratch_types=[
        pltpu.VMEM_SHARED((N, D), jnp.float32),
        pltpu.VMEM((EPT,), jnp.int32),
        pltpu.VMEM((2, GRP, CH), jnp.int32),
        pltpu.VMEM((NB, CH, D), jnp.float32),
        pltpu.SemaphoreType.DMA((NB,)),
        pltpu.SemaphoreType.DMA((NB,)),
        pltpu.SemaphoreType.DMA((2,)),
    ],
    compiler_params=_sc_params,
)
def _edge_kernel(src_hbm, dst_hbm, g_hbm, parts_hbm, acc, sidx, didx, rows,
                 gsem, ssem, isem):
    cid = lax.axis_index("c")
    sid = lax.axis_index("s")
    widx = cid * NS + sid
    zeros16 = jnp.zeros((16,), jnp.float32)
    r0 = rows.at[0]

    @pl.loop(0, CH)
    def _(r):
        @pl.loop(0, D // 16)
        def _(c):
            r0.at[r][pl.ds(c * 16, 16)] = zeros16

    @pl.loop(0, ROWS_PT // CH)
    def _(i):
        pltpu.sync_copy(r0, acc.at[pl.ds(sid * ROWS_PT + i * CH, CH)])

    pltpu.sync_copy(
        r0.at[pl.ds(0, ROWS_PT % CH)],
        acc.at[pl.ds(sid * ROWS_PT + (ROWS_PT // CH) * CH, ROWS_PT % CH)],
    )

    plsc.subcore_barrier()

    tb = widx * CPT
    nch = jnp.minimum(CPT, NCH - tb)
    ngrp = nch // GRP
    nsb = ngrp // 2

    @pl.when(widx < NW - 1)
    def _():
        pltpu.sync_copy(src_hbm.at[pl.ds(widx * EPT, EPT)], sidx)

    @pl.when(widx == NW - 1)
    def _():
        pltpu.sync_copy(
            src_hbm.at[pl.ds((NW - 1) * EPT, E - (NW - 1) * EPT)],
            sidx.at[pl.ds(0, E - (NW - 1) * EPT)],
        )

    for q in range(2):
        pltpu.async_copy(dst_hbm.at[pl.ds(tb + q * GRP, GRP)], didx.at[q],
                         isem.at[q])
    for c in range(NB):
        pltpu.async_copy(g_hbm.at[sidx.at[pl.ds(c * CH, CH)]], rows.at[c],
                         gsem.at[c])

    @pl.loop(0, nsb)
    def _(s):
        for q in range(2):
            g = s * 2 + q
            grow = tb + g * GRP
            pltpu.make_async_copy(
                dst_hbm.at[pl.ds(grow, GRP)], didx.at[q], isem.at[q]
            ).wait()
            for r in range(GRP // NB):
                for c in range(NB):
                    k = g * GRP + r * NB + c
                    pltpu.make_async_copy(
                        g_hbm.at[sidx.at[pl.ds(k * CH, CH)]], rows.at[c],
                        gsem.at[c],
                    ).wait()
                    pltpu.async_copy(
                        rows.at[c], acc.at[didx.at[q].at[r * NB + c]],
                        ssem.at[c], add=True,
                    )
                for c in range(NB):
                    k = g * GRP + r * NB + c
                    pltpu.make_async_copy(
                        rows.at[c], acc.at[didx.at[q].at[r * NB + c]],
                        ssem.at[c],
                    ).wait()

                    @pl.when(k + NB < nch)
                    def _():
                        pltpu.async_copy(
                            g_hbm.at[sidx.at[pl.ds((k + NB) * CH, CH)]],
                            rows.at[c], gsem.at[c],
                        )

            @pl.when(g + 2 < ngrp)
            def _():
                pltpu.async_copy(
                    dst_hbm.at[pl.ds(grow + 2 * GRP, GRP)], didx.at[q],
                    isem.at[q],
                )

    @pl.when(ngrp > nsb * 2)
    def _():
        g = nsb * 2
        grow = tb + g * GRP
        pltpu.make_async_copy(
            dst_hbm.at[pl.ds(grow, GRP)], didx.at[0], isem.at[0]
        ).wait()
        for r in range(GRP // NB):
            for c in range(NB):
                k = g * GRP + r * NB + c
                pltpu.make_async_copy(
                    g_hbm.at[sidx.at[pl.ds(k * CH, CH)]], rows.at[c],
                    gsem.at[c],
                ).wait()
                pltpu.async_copy(
                    rows.at[c], acc.at[didx.at[0].at[r * NB + c]],
                    ssem.at[c], add=True,
                )
            for c in range(NB):
                k = g * GRP + r * NB + c
                pltpu.make_async_copy(
                    rows.at[c], acc.at[didx.at[0].at[r * NB + c]],
                    ssem.at[c],
                ).wait()

                @pl.when(k + NB < nch)
                def _():
                    pltpu.async_copy(
                        g_hbm.at[sidx.at[pl.ds((k + NB) * CH, CH)]],
                        rows.at[c], gsem.at[c],
                    )

    plsc.subcore_barrier()

    @pl.when(sid < NS - 1)
    def _():
        pltpu.sync_copy(
            acc.at[pl.ds(sid * CO, CO)],
            parts_hbm.at[cid].at[pl.ds(sid * CO, CO)],
        )

    @pl.when(sid == NS - 1)
    def _():
        pltpu.sync_copy(
            acc.at[pl.ds((NS - 1) * CO, N - (NS - 1) * CO)],
            parts_hbm.at[cid].at[pl.ds((NS - 1) * CO, N - (NS - 1) * CO)],
        )


def _gw_body(x_ref, w_ref, degp_ref, g_ref):
    deg = jnp.sum(degp_ref[...], axis=1) + 1.0
    dinv = lax.rsqrt(deg)
    h = jnp.dot(
        x_ref[...].astype(jnp.bfloat16),
        w_ref[...].astype(jnp.bfloat16),
        preferred_element_type=jnp.float32,
    )
    g_ref[...] = h * dinv[:, None]


def _final_body(p_ref, g_ref, degp_ref, b_ref, o_ref):
    deg = jnp.sum(degp_ref[...], axis=1) + 1.0
    dinv = lax.rsqrt(deg)
    p = p_ref[...]
    s = p[0] + p[1] + g_ref[...]
    o_ref[...] = jnp.maximum(s * dinv[:, None] + b_ref[...], 0.0)


def kernel(x, edge_index, W, b):
    src_flat = edge_index[0]
    dst64 = edge_index[1].reshape(NCH, CH)
    dst128 = edge_index[1].reshape(K1CH, 128)

    degp = _deg_kernel(dst128).T

    g = pl.pallas_call(
        _gw_body,
        grid=(N // BLK,),
        in_specs=[
            pl.BlockSpec((BLK, D), lambda i: (i, 0)),
            pl.BlockSpec((D, D), lambda i: (0, 0)),
            pl.BlockSpec((BLK, NW), lambda i: (i, 0)),
        ],
        out_specs=pl.BlockSpec((BLK, D), lambda i: (i, 0)),
        out_shape=jax.ShapeDtypeStruct((N, D), jnp.float32),
    )(x, W, degp)

    parts = _edge_kernel(src_flat, dst64, g)

    out = pl.pallas_call(
        _final_body,
        grid=(N // BLK,),
        in_specs=[
            pl.BlockSpec((NC, BLK, D), lambda i: (0, i, 0)),
            pl.BlockSpec((BLK, D), lambda i: (i, 0)),
            pl.BlockSpec((BLK, NW), lambda i: (i, 0)),
            pl.BlockSpec((1, D), lambda i: (0, 0)),
        ],
        out_specs=pl.BlockSpec((BLK, D), lambda i: (i, 0)),
        out_shape=jax.ShapeDtypeStruct((N, D), jnp.float32),
    )(parts, g, degp, b.reshape(1, D))

    return out


# speedup vs baseline: 44.0514x; 1.0017x over previous
import dataclasses
import functools

import jax
import jax.numpy as jnp
from jax import lax
from jax.experimental import pallas as pl
from jax.experimental.pallas import tpu as pltpu
from jax.experimental.pallas import tpu_sc as plsc

N = 10000
E = 320000
D = 128

NC = 2
NS = 16
NW = NC * NS
CH = 64
NCH = E // CH
CPT = 160
EPT = CPT * CH
GRP = 8
NGRP = CPT // GRP
NB = 4
ROWS_PT = N // NS
CO = 624
K1CH = E // 128
K1CPT = 80
BLK = 2000

_mesh = plsc.VectorSubcoreMesh(
    core_axis_name="c", subcore_axis_name="s", num_cores=NC, num_subcores=NS
)

_sc_params = pltpu.CompilerParams()
if "needs_layout_passes" in pltpu.CompilerParams.__dataclass_fields__:
    _sc_params = dataclasses.replace(_sc_params, needs_layout_passes=False)
if "use_tc_tiling_on_sc" in pltpu.CompilerParams.__dataclass_fields__:
    _sc_params = dataclasses.replace(_sc_params, use_tc_tiling_on_sc=True)


@functools.partial(
    pl.kernel,
    out_type=jax.ShapeDtypeStruct((NW, N), jnp.float32),
    mesh=_mesh,
    scratch_types=[
        pltpu.VMEM((N,), jnp.float32),
        pltpu.VMEM((K1CPT, 128), jnp.int32),
    ],
    compiler_params=_sc_params,
)
def _deg_kernel(dst_hbm, degp_hbm, hist, idx2d):
    wid = lax.axis_index("s") * NC + lax.axis_index("c")
    zeros16 = jnp.zeros((16,), jnp.float32)

    @pl.loop(0, N // 16)
    def _(j):
        hist[pl.ds(j * 16, 16)] = zeros16

    nch = jnp.minimum(K1CPT, K1CH - wid * K1CPT)

    @pl.when(wid < NW - 1)
    def _():
        pltpu.sync_copy(dst_hbm.at[pl.ds(wid * K1CPT, K1CPT)], idx2d)

    @pl.when(wid == NW - 1)
    def _():
        pltpu.sync_copy(
            dst_hbm.at[pl.ds((NW - 1) * K1CPT, K1CH - (NW - 1) * K1CPT)],
            idx2d.at[pl.ds(0, K1CH - (NW - 1) * K1CPT)],
        )

    ones16 = jnp.ones((16,), jnp.float32)

    @pl.loop(0, nch)
    def _(j):
        @pl.loop(0, 128 // 16)
        def _(c):
            v = idx2d.at[j][pl.ds(c * 16, 16)]
            plsc.addupdate_scatter(hist, [v], ones16)

    pltpu.sync_copy(hist, degp_hbm.at[wid])


@functools.partial(
    pl.kernel,
    out_type=jax.ShapeDtypeStruct((NC, N, D), jnp.float32),
    mesh=_mesh,
    scratch_types=[
        pltpu.VMEM_SHARED((N, D), jnp.float32),
        pltpu.VMEM((EPT,), jnp.int32),
        pltpu.VMEM((2, GRP, CH), jnp.int32),
        pltpu.VMEM((NB, CH, D), jnp.float32),
        pltpu.SemaphoreType.DMA((NB,)),
        pltpu.SemaphoreType.DMA((NB,)),
        pltpu.SemaphoreType.DMA((2,)),
    ],
    compiler_params=_sc_params,
)
def _edge_kernel(src_hbm, dst_hbm, g_hbm, parts_hbm, acc, sidx, didx, rows,
                 gsem, ssem, isem):
    cid = lax.axis_index("c")
    sid = lax.axis_index("s")
    widx = cid * NS + sid
    zeros16 = jnp.zeros((16,), jnp.float32)
    r0 = rows.at[0]

    @pl.loop(0, CH)
    def _(r):
        @pl.loop(0, D // 16)
        def _(c):
            r0.at[r][pl.ds(c * 16, 16)] = zeros16

    @pl.loop(0, ROWS_PT // CH)
    def _(i):
        pltpu.sync_copy(r0, acc.at[pl.ds(sid * ROWS_PT + i * CH, CH)])

    pltpu.sync_copy(
        r0.at[pl.ds(0, ROWS_PT % CH)],
        acc.at[pl.ds(sid * ROWS_PT + (ROWS_PT // CH) * CH, ROWS_PT % CH)],
    )

    plsc.subcore_barrier()

    tb = widx * CPT
    nch = jnp.minimum(CPT, NCH - tb)
    ngrp = nch // GRP
    nsb = ngrp // 2

    @pl.when(widx < NW - 1)
    def _():
        pltpu.sync_copy(src_hbm.at[pl.ds(widx * EPT, EPT)], sidx)

    @pl.when(widx == NW - 1)
    def _():
        pltpu.sync_copy(
            src_hbm.at[pl.ds((NW - 1) * EPT, E - (NW - 1) * EPT)],
            sidx.at[pl.ds(0, E - (NW - 1) * EPT)],
        )

    for q in range(2):
        pltpu.async_copy(dst_hbm.at[pl.ds(tb + q * GRP, GRP)], didx.at[q],
                         isem.at[q])
    for c in range(NB):
        pltpu.async_copy(g_hbm.at[sidx.at[pl.ds(c * CH, CH)]], rows.at[c],
                         gsem.at[c])

    @pl.loop(0, nsb)
    def _(s):
        for q in range(2):
            g = s * 2 + q
            grow = tb + g * GRP
            pltpu.make_async_copy(
                dst_hbm.at[pl.ds(grow, GRP)], didx.at[q], isem.at[q]
            ).wait()
            for r in range(GRP // NB):
                for c in range(NB):
                    k = g * GRP + r * NB + c
                    pltpu.make_async_copy(
                        g_hbm.at[sidx.at[pl.ds(k * CH, CH)]], rows.at[c],
                        gsem.at[c],
                    ).wait()
                    pltpu.async_copy(
                        rows.at[c], acc.at[didx.at[q].at[r * NB + c]],
                        ssem.at[c], add=True,
                    )
                for c in range(NB):
                    k = g * GRP + r * NB + c
                    pltpu.make_async_copy(
                        rows.at[c], acc.at[didx.at[q].at[r * NB + c]],
                        ssem.at[c],
                    ).wait()

                    @pl.when(k + NB < nch)
                    def _():
                        pltpu.async_copy(
                            g_hbm.at[sidx.at[pl.ds((k + NB) * CH, CH)]],
                            rows.at[c], gsem.at[c],
                        )

            @pl.when(g + 2 < ngrp)
            def _():
                pltpu.async_copy(
                    dst_hbm.at[pl.ds(grow + 2 * GRP, GRP)], didx.at[q],
                    isem.at[q],
                )

    @pl.when(ngrp > nsb * 2)
    def _():
        g = nsb * 2
        grow = tb + g * GRP
        pltpu.make_async_copy(
            dst_hbm.at[pl.ds(grow, GRP)], didx.at[0], isem.at[0]
        ).wait()
        for r in range(GRP // NB):
            for c in range(NB):
                k = g * GRP + r * NB + c
                pltpu.make_async_copy(
                    g_hbm.at[sidx.at[pl.ds(k * CH, CH)]], rows.at[c],
                    gsem.at[c],
                ).wait()
                pltpu.async_copy(
                    rows.at[c], acc.at[didx.at[0].at[r * NB + c]],
                    ssem.at[c], add=True,
                )
            for c in range(NB):
                k = g * GRP + r * NB + c
                pltpu.make_async_copy(
                    rows.at[c], acc.at[didx.at[0].at[r * NB + c]],
                    ssem.at[c],
                ).wait()

                @pl.when(k + NB < nch)
                def _():
                    pltpu.async_copy(
                        g_hbm.at[sidx.at[pl.ds((k + NB) * CH, CH)]],
                        rows.at[c], gsem.at[c],
                    )

    plsc.subcore_barrier()

    @pl.when(sid < NS - 1)
    def _():
        pltpu.sync_copy(
            acc.at[pl.ds(sid * CO, CO)],
            parts_hbm.at[cid].at[pl.ds(sid * CO, CO)],
        )

    @pl.when(sid == NS - 1)
    def _():
        pltpu.sync_copy(
            acc.at[pl.ds((NS - 1) * CO, N - (NS - 1) * CO)],
            parts_hbm.at[cid].at[pl.ds((NS - 1) * CO, N - (NS - 1) * CO)],
        )


def _gw_body(x_ref, w_ref, degp_ref, g_ref):
    deg = jnp.sum(degp_ref[...], axis=1) + 1.0
    dinv = lax.rsqrt(deg)
    h = jnp.dot(
        x_ref[...].astype(jnp.bfloat16),
        w_ref[...].astype(jnp.bfloat16),
        preferred_element_type=jnp.float32,
    )
    g_ref[...] = h * dinv[:, None]


def _final_body(p_ref, g_ref, degp_ref, b_ref, o_ref):
    deg = jnp.sum(degp_ref[...], axis=1) + 1.0
    dinv = lax.rsqrt(deg)
    p = p_ref[...]
    s = p[0] + p[1] + g_ref[...]
    o_ref[...] = jnp.maximum(s * dinv[:, None] + b_ref[...], 0.0)


def kernel(x, edge_index, W, b):
    src_flat = edge_index[0]
    dst64 = edge_index[1].reshape(NCH, CH)
    dst128 = edge_index[1].reshape(K1CH, 128)

    degp = _deg_kernel(dst128).T

    g = pl.pallas_call(
        _gw_body,
        grid=(N // BLK,),
        in_specs=[
            pl.BlockSpec((BLK, D), lambda i: (i, 0)),
            pl.BlockSpec((D, D), lambda i: (0, 0)),
            pl.BlockSpec((BLK, NW), lambda i: (i, 0)),
        ],
        out_specs=pl.BlockSpec((BLK, D), lambda i: (i, 0)),
        out_shape=jax.ShapeDtypeStruct((N, D), jnp.float32),
    )(x, W, degp)

    parts = _edge_kernel(src_flat, dst64, g)

    out = pl.pallas_call(
        _final_body,
        grid=(N // BLK,),
        in_specs=[
            pl.BlockSpec((NC, BLK, D), lambda i: (0, i, 0)),
            pl.BlockSpec((BLK, D), lambda i: (i, 0)),
            pl.BlockSpec((BLK, NW), lambda i: (i, 0)),
            pl.BlockSpec((1, D), lambda i: (0, 0)),
        ],
        out_specs=pl.BlockSpec((BLK, D), lambda i: (i, 0)),
        out_shape=jax.ShapeDtypeStruct((N, D), jnp.float32),
    )(parts, g, degp, b.reshape(1, D))

    return out
